# baseline (speedup 1.0000x reference)
import functools

import jax
import jax.numpy as jnp
from jax import lax
from jax.experimental import pallas as pl
from jax.experimental.pallas import tpu as pltpu
from jax.experimental.pallas import tpu_sc as plsc

_VOCAB = 1000000
_EMBED = 64
_NC = 2
_NS = 16
_NW = _NC * _NS
_CH = 100
_NBUF = 4


def _make_emb_kernel(n_b: int):
    mesh = plsc.VectorSubcoreMesh(core_axis_name="c", subcore_axis_name="s")

    @functools.partial(
        pl.kernel,
        mesh=mesh,
        compiler_params=pltpu.CompilerParams(use_tc_tiling_on_sc=False),
        out_type=jax.ShapeDtypeStruct((_NW * n_b, 200, _EMBED), jnp.float32),
        scratch_types=[
            pltpu.VMEM((n_b, 2, _CH), jnp.int32),
            pltpu.VMEM((_NBUF, 2 * _CH, _EMBED), jnp.float32),
            pltpu.SemaphoreType.DMA((_NBUF, 2)),
            pltpu.SemaphoreType.DMA((_NBUF,)),
        ],
    )
    def emb(idx_hbm, table_hbm, out_hbm, idx_v, rows, gsem, ssem):
        wid = lax.axis_index("s") * _NC + lax.axis_index("c")
        base = wid * n_b
        pltpu.sync_copy(idx_hbm.at[wid], idx_v)

        def gather(b, f):
            pltpu.async_copy(
                table_hbm.at[idx_v.at[b, 0]],
                rows.at[f, pl.ds(0, _CH)],
                gsem.at[f, 0],
            )
            pltpu.async_copy(
                table_hbm.at[idx_v.at[b, 1]],
                rows.at[f, pl.ds(_CH, _CH)],
                gsem.at[f, 1],
            )

        def gwait(b, f):
            pltpu.make_async_copy(
                table_hbm.at[idx_v.at[b, 0]],
                rows.at[f, pl.ds(0, _CH)],
                gsem.at[f, 0],
            ).wait()
            pltpu.make_async_copy(
                table_hbm.at[idx_v.at[b, 1]],
                rows.at[f, pl.ds(_CH, _CH)],
                gsem.at[f, 1],
            ).wait()

        def store(b, f):
            pltpu.async_copy(rows.at[f], out_hbm.at[base + b], ssem.at[f])

        def swait(b, f):
            pltpu.make_async_copy(
                rows.at[f], out_hbm.at[base + b], ssem.at[f]
            ).wait()

        for f in range(_NBUF):
            gather(f, f)

        def body(i, _):
            b0 = i * _NBUF
            for f in range(_NBUF):
                b = b0 + f
                gwait(b, f)
                store(b, f)
                swait(b, f)

                @pl.when(b + _NBUF < n_b)
                def _():
                    gather(b + _NBUF, f)

            return 0

        lax.fori_loop(0, n_b // _NBUF, body, 0)

    return emb


def kernel(input, table):
    bsz, seq = input.shape
    n_b = bsz // _NW
    idx = input.reshape(_NW, n_b, 2, _CH).astype(jnp.int32)
    t2 = jax.lax.optimization_barrier(table.reshape(_VOCAB // 2, 2 * _EMBED))
    t3 = t2.reshape(_VOCAB, _EMBED)
    return _make_emb_kernel(n_b)(idx, t3)
